# baseline (speedup 1.0000x reference)
import jax
import jax.numpy as jnp
from jax.experimental import pallas as pl
from jax.experimental.pallas import tpu as pltpu

_SCORE_THRESH = 0.05
_NMS_THRESH = 0.5
_DETS = 100
_N = 5000
_LANES = 128
_VLANES = 125
_ROWS = 40
_K = 4 * _VLANES
_NEG_INF = float("-inf")
_HI = jax.lax.Precision.HIGHEST


def _nms_body(bv_ref, bts_ref, s0_ref, raw0_ref, kb_ref, ks_ref, keep_ref):
    bview = bv_ref[...]
    kio = jax.lax.broadcasted_iota(jnp.int32, (_K, _LANES), 0)
    lio = jax.lax.broadcasted_iota(jnp.int32, (_K, _LANES), 1)

    def plane(j):
        g = jnp.where(kio == 4 * lio + j, 1.0, 0.0)
        return jax.lax.dot_general(bview, g, (((1,), (0,)), ((), ())),
                                   precision=_HI)

    x0 = plane(0)
    y0 = plane(1)
    x1 = plane(2)
    y1 = plane(3)
    area = (x1 - x0) * (y1 - y0)
    rows = jax.lax.broadcasted_iota(jnp.int32, (_ROWS, _LANES), 0)
    lanes = jax.lax.broadcasted_iota(jnp.int32, (_ROWS, _LANES), 1)
    idxf = (rows * _VLANES + lanes).astype(jnp.float32)
    raw0 = raw0_ref[0]

    def body(i, s):
        m = jnp.max(s)
        cand = jnp.where(s == m, idxf, 8192.0)
        idx = jnp.min(cand).astype(jnp.int32)

        base = idx * 4
        bx0 = bts_ref[base]
        by0 = bts_ref[base + 1]
        bx1 = bts_ref[base + 2]
        by1 = bts_ref[base + 3]

        ltx = jnp.maximum(bx0, x0)
        lty = jnp.maximum(by0, y0)
        rbx = jnp.minimum(bx1, x1)
        rby = jnp.minimum(by1, y1)
        wx = jnp.maximum(rbx - ltx, 0.0)
        wy = jnp.maximum(rby - lty, 0.0)
        inter = wx * wy
        area_b = (bx1 - bx0) * (by1 - by0)
        iou = inter / (area_b + area - inter + 1e-9)
        snew = jnp.where(iou > _NMS_THRESH, _NEG_INF, s)

        keep_ref[i] = idx
        ks_ref[i] = jnp.where(m == _NEG_INF, raw0, m)
        kb_ref[i, 0] = bx0
        kb_ref[i, 1] = by0
        kb_ref[i, 2] = bx1
        kb_ref[i, 3] = by1
        return snew

    jax.lax.fori_loop(0, _DETS, body, s0_ref[...], unroll=False)


@jax.jit
def _nms(boxes, scores):
    bview = boxes.reshape(_ROWS, _K)
    bts = boxes.reshape(4 * _N)
    s0 = jnp.where(scores > _SCORE_THRESH, scores, _NEG_INF)
    s0 = jnp.pad(s0.reshape(_ROWS, _VLANES), ((0, 0), (0, _LANES - _VLANES)),
                 constant_values=_NEG_INF)
    raw0 = scores[0:1]

    smem = pl.BlockSpec(memory_space=pltpu.SMEM)
    vmem = pl.BlockSpec(memory_space=pltpu.VMEM)
    kb, ks, keep = pl.pallas_call(
        _nms_body,
        out_shape=[
            jax.ShapeDtypeStruct((_DETS, 4), jnp.float32),
            jax.ShapeDtypeStruct((_DETS,), jnp.float32),
            jax.ShapeDtypeStruct((_DETS,), jnp.int32),
        ],
        in_specs=[vmem, smem, vmem, smem],
        out_specs=[smem, smem, smem],
    )(bview, bts, s0, raw0)
    return kb, ks, keep


def kernel(boxes, scores):
    return _nms(boxes, scores)


# speedup vs baseline: 1.4390x; 1.4390x over previous
import jax
import jax.numpy as jnp
from jax import lax
from jax.experimental import pallas as pl
from jax.experimental.pallas import tpu as pltpu

_SCORE_THRESH = 0.05
_NMS_THRESH = 0.5
_DETS = 100
_N = 5000
_LANES = 128
_ROWS = 40
_PAD = _ROWS * _LANES
_NEG_INF = float("-inf")
_BIG = float(2 * _PAD)


def _nms_body(bt_ref, bts_ref, s0_ref, raw0_ref, kb_ref, ks_ref, keep_ref):
    x0 = bt_ref[0]
    y0 = bt_ref[1]
    x1 = bt_ref[2]
    y1 = bt_ref[3]
    area = (x1 - x0) * (y1 - y0)
    rows = jax.lax.broadcasted_iota(jnp.int32, (_ROWS, _LANES), 0)
    lanes = jax.lax.broadcasted_iota(jnp.int32, (_ROWS, _LANES), 1)
    idxf = (rows * _LANES + lanes).astype(jnp.float32)
    raw0 = raw0_ref[0]
    s_init = s0_ref[...]
    m_init = jnp.max(s_init)

    def resolve(s, m):
        hot = s == m
        i_f = jnp.min(jnp.where(hot, idxf, _BIG))
        imax_f = jnp.max(jnp.where(hot, idxf, -1.0))
        m2 = jnp.max(jnp.where(hot, _NEG_INF, s))
        return m, i_f, imax_f, m2

    def body(k, carry):
        s, m = carry
        vals = resolve(s, m)
        m, i_f, imax_f, m2 = lax.cond(
            vals[1] >= _BIG,
            lambda: resolve(s, jnp.max(s)),
            lambda: vals,
        )
        idx = i_f.astype(jnp.int32)

        base = idx * 4
        bx0 = bts_ref[base]
        by0 = bts_ref[base + 1]
        bx1 = bts_ref[base + 2]
        by1 = bts_ref[base + 3]

        ltx = jnp.maximum(bx0, x0)
        lty = jnp.maximum(by0, y0)
        rbx = jnp.minimum(bx1, x1)
        rby = jnp.minimum(by1, y1)
        wx = jnp.maximum(rbx - ltx, 0.0)
        wy = jnp.maximum(rby - lty, 0.0)
        inter = wx * wy
        area_b = (bx1 - bx0) * (by1 - by0)
        iou = inter / (area_b + area - inter + 1e-9)
        snew = jnp.where(iou > _NMS_THRESH, _NEG_INF, s)

        keep_ref[k] = idx
        ks_ref[k] = jnp.where(m == _NEG_INF, raw0, m)
        kb_ref[k, 0] = bx0
        kb_ref[k, 1] = by0
        kb_ref[k, 2] = bx1
        kb_ref[k, 3] = by1

        mnext = jnp.where(imax_f != i_f, m, m2)
        return snew, mnext

    lax.fori_loop(0, _DETS, body, (s_init, m_init), unroll=False)


@jax.jit
def _nms(boxes, scores):
    bt = jnp.pad(boxes.T, ((0, 0), (0, _PAD - _N))).reshape(4, _ROWS, _LANES)
    s0 = jnp.where(scores > _SCORE_THRESH, scores, _NEG_INF)
    s0 = jnp.pad(s0, (0, _PAD - _N), constant_values=_NEG_INF)
    s0 = s0.reshape(_ROWS, _LANES)
    bts = boxes.reshape(4 * _N)
    raw0 = scores[0:1]

    smem = pl.BlockSpec(memory_space=pltpu.SMEM)
    vmem = pl.BlockSpec(memory_space=pltpu.VMEM)
    kb, ks, keep = pl.pallas_call(
        _nms_body,
        out_shape=[
            jax.ShapeDtypeStruct((_DETS, 4), jnp.float32),
            jax.ShapeDtypeStruct((_DETS,), jnp.float32),
            jax.ShapeDtypeStruct((_DETS,), jnp.int32),
        ],
        in_specs=[vmem, smem, vmem, smem],
        out_specs=[smem, smem, smem],
    )(bt, bts, s0, raw0)
    return kb, ks, keep


def kernel(boxes, scores):
    return _nms(boxes, scores)
